# Initial kernel scaffold; baseline (speedup 1.0000x reference)
import jax
import jax.numpy as jnp
from jax.experimental import pallas as pl


def kernel(x, edge_index, edge_attr, batch, W1, W2, We, Wout, bout):
    raise NotImplementedError("write your pallas kernel here")


# speedup vs baseline: 2.1813x; 2.1813x over previous
import functools

import jax
import jax.numpy as jnp
from jax import lax
from jax.experimental import pallas as pl
from jax.experimental.pallas import tpu as pltpu
from jax.experimental.pallas import tpu_sc as plsc

_N = 10000
_E = 320000
_D = 128
_DE = 16
_B = 512
_OUT = 12

_NC = 1
_NS = 16
_CHUNK = 80
_EPC = _E // _NC
_EPW = _EPC // _NS
_NCHUNK = _EPW // _CHUNK
_RSTRIPE = 624
_RCHUNKS = 8


def _sc_agg_body(x_hbm, ea_hbm, src_hbm, dst_hbm, zx_hbm, ze_hbm, iota_hbm,
                 outx_hbm, oute_hbm,
                 src_v, dst_v, rows_v, ea_v, accx_s, acce_s, gsem):
    c = lax.axis_index("c")
    s = lax.axis_index("s")

    pltpu.sync_copy(zx_hbm, rows_v)
    pltpu.sync_copy(ze_hbm, ea_v)

    def zstep(j, carry):
        base = s * _RSTRIPE + j * _CHUNK
        pltpu.sync_copy(iota_hbm.at[pl.ds(base, _CHUNK)], src_v)
        pltpu.sync_copy(rows_v, accx_s.at[src_v])
        pltpu.sync_copy(ea_v, acce_s.at[src_v])
        return carry

    lax.fori_loop(0, _RCHUNKS, zstep, 0)
    plsc.subcore_barrier()

    e0 = c * _EPC + s * _EPW

    def step(k, carry):
        base = e0 + k * _CHUNK
        pltpu.sync_copy(src_hbm.at[pl.ds(base, _CHUNK)], src_v)
        pltpu.sync_copy(dst_hbm.at[pl.ds(base, _CHUNK)], dst_v)
        pltpu.async_copy(x_hbm.at[src_v], rows_v, gsem).wait()
        pltpu.sync_copy(ea_hbm.at[pl.ds(base, _CHUNK)], ea_v)
        pltpu.sync_copy(rows_v, accx_s.at[dst_v], add=True)
        pltpu.sync_copy(ea_v, acce_s.at[dst_v], add=True)
        return carry

    lax.fori_loop(0, _NCHUNK, step, 0)
    plsc.subcore_barrier()

    def wstep(j, carry):
        base = s * _RSTRIPE + j * _CHUNK
        pltpu.sync_copy(iota_hbm.at[pl.ds(base, _CHUNK)], src_v)
        pltpu.async_copy(accx_s.at[src_v], rows_v, gsem).wait()
        pltpu.sync_copy(rows_v, outx_hbm.at[pl.ds(c * _N + base, _CHUNK)])
        pltpu.async_copy(acce_s.at[src_v], ea_v, gsem).wait()
        pltpu.sync_copy(ea_v, oute_hbm.at[pl.ds(c * _N + base, _CHUNK)])
        return carry

    lax.fori_loop(0, _RCHUNKS, wstep, 0)


@functools.cache
def _sc_agg():
    return functools.partial(
        pl.kernel,
        mesh=plsc.VectorSubcoreMesh(core_axis_name="c", subcore_axis_name="s",
                                    num_cores=_NC),
        compiler_params=pltpu.CompilerParams(use_tc_tiling_on_sc=False),
        out_type=[jax.ShapeDtypeStruct((_NC * _N, _D), jnp.float32),
                  jax.ShapeDtypeStruct((_NC * _N, _DE), jnp.float32)],
        scratch_types=[
            pltpu.VMEM((_CHUNK,), jnp.int32),
            pltpu.VMEM((_CHUNK,), jnp.int32),
            pltpu.VMEM((_CHUNK, _D), jnp.float32),
            pltpu.VMEM((_CHUNK, _DE), jnp.float32),
            pltpu.VMEM_SHARED((_N, _D), jnp.float32),
            pltpu.VMEM_SHARED((_N, _DE), jnp.float32),
            pltpu.SemaphoreType.DMA,
        ],
    )(_sc_agg_body)


def _post_body(x_ref, w1_ref, w2_ref, we_ref, wout_ref, bout_ref,
               aggx_ref, agge_ref, batch_ref, bshift_ref, o_ref):
    aggx = aggx_ref[pl.ds(0, _N), :]
    agge = agge_ref[pl.ds(0, _N), :]
    for c in range(1, _NC):
        aggx = aggx + aggx_ref[pl.ds(c * _N, _N), :]
        agge = agge + agge_ref[pl.ds(c * _N, _N), :]
    h = (jnp.dot(x_ref[...], w1_ref[...], preferred_element_type=jnp.float32)
         + jnp.dot(aggx, w2_ref[...], preferred_element_type=jnp.float32)
         + jnp.dot(agge, we_ref[...], preferred_element_type=jnp.float32))
    h = jnp.maximum(h, 0.0)
    b = batch_ref[...]
    bs = bshift_ref[...]
    gid = lax.broadcasted_iota(jnp.int32, (1, _N), 1)
    keep = jnp.logical_and(b == bs, gid != _N - 1)
    gids = lax.broadcasted_iota(jnp.int32, (_B, _N), 0)
    pt = jnp.where(jnp.logical_and(gids == b, keep), 1.0, 0.0)
    hg = jnp.dot(pt, h, preferred_element_type=jnp.float32)
    cnt = jnp.sum(pt, axis=1, keepdims=True)
    hg = hg / jnp.maximum(cnt, 1.0)
    o_ref[...] = (jnp.dot(hg, wout_ref[...], preferred_element_type=jnp.float32)
                  + bout_ref[...])


def _post_call(interpret=False):
    return pl.pallas_call(
        _post_body,
        out_shape=jax.ShapeDtypeStruct((_B, _OUT), jnp.float32),
        interpret=interpret,
    )


def kernel(x, edge_index, edge_attr, batch, W1, W2, We, Wout, bout):
    src = edge_index[0].astype(jnp.int32)
    dst = edge_index[1].astype(jnp.int32)
    batch32 = batch.astype(jnp.int32)
    bshift = jnp.concatenate([batch32[:1], batch32[:-1]])
    zx = jnp.zeros((_CHUNK, _D), jnp.float32)
    ze = jnp.zeros((_CHUNK, _DE), jnp.float32)
    iota = jnp.arange(_NC * _N, dtype=jnp.int32)
    aggx, agge = _sc_agg()(x, edge_attr, src, dst, zx, ze, iota)
    return _post_call()(
        x, W1, W2, We, Wout, bout.reshape(1, _OUT),
        aggx, agge, batch32.reshape(1, _N), bshift.reshape(1, _N))


# speedup vs baseline: 3.5752x; 1.6391x over previous
import functools

import jax
import jax.numpy as jnp
from jax import lax
from jax.experimental import pallas as pl
from jax.experimental.pallas import tpu as pltpu
from jax.experimental.pallas import tpu_sc as plsc

_N = 10000
_E = 320000
_D = 128
_DE = 16
_B = 512
_OUT = 12

_NC = 2
_NS = 16
_CHUNK = 80
_EPC = _E // _NC
_EPW = _EPC // _NS
_NCHUNK = _EPW // _CHUNK
_RSTRIPE = 624
_RCHUNKS = 8


def _sc_agg_body(x_hbm, ea_hbm, src_hbm, dst_hbm, zx_hbm, ze_hbm, iota_hbm,
                 outx_hbm, oute_hbm,
                 src_v, dst_v, rows_v, ea_v, accx_s, acce_s, gsem):
    c = lax.axis_index("c")
    s = lax.axis_index("s")

    pltpu.sync_copy(zx_hbm, rows_v)
    pltpu.sync_copy(ze_hbm, ea_v)

    def zstep(j, carry):
        base = s * _RSTRIPE + j * _CHUNK
        pltpu.sync_copy(iota_hbm.at[pl.ds(base, _CHUNK)], src_v)
        pltpu.sync_copy(rows_v, accx_s.at[src_v])
        pltpu.sync_copy(ea_v, acce_s.at[src_v])
        return carry

    lax.fori_loop(0, _RCHUNKS, zstep, 0)
    plsc.subcore_barrier()

    e0 = c * _EPC + s * _EPW

    def step(k, carry):
        base = e0 + k * _CHUNK
        pltpu.sync_copy(src_hbm.at[pl.ds(base, _CHUNK)], src_v)
        pltpu.sync_copy(dst_hbm.at[pl.ds(base, _CHUNK)], dst_v)
        pltpu.async_copy(x_hbm.at[src_v], rows_v, gsem).wait()
        pltpu.sync_copy(ea_hbm.at[pl.ds(base, _CHUNK)], ea_v)
        pltpu.sync_copy(rows_v, accx_s.at[dst_v], add=True)
        pltpu.sync_copy(ea_v, acce_s.at[dst_v], add=True)
        return carry

    lax.fori_loop(0, _NCHUNK, step, 0)
    plsc.subcore_barrier()

    def wstep(j, carry):
        base = s * _RSTRIPE + j * _CHUNK
        pltpu.sync_copy(iota_hbm.at[pl.ds(base, _CHUNK)], src_v)
        pltpu.async_copy(accx_s.at[src_v], rows_v, gsem).wait()
        pltpu.sync_copy(rows_v, outx_hbm.at[pl.ds(c * _N + base, _CHUNK)])
        pltpu.async_copy(acce_s.at[src_v], ea_v, gsem).wait()
        pltpu.sync_copy(ea_v, oute_hbm.at[pl.ds(c * _N + base, _CHUNK)])
        return carry

    lax.fori_loop(0, _RCHUNKS, wstep, 0)


@functools.cache
def _sc_agg():
    return functools.partial(
        pl.kernel,
        mesh=plsc.VectorSubcoreMesh(core_axis_name="c", subcore_axis_name="s",
                                    num_cores=_NC),
        compiler_params=pltpu.CompilerParams(use_tc_tiling_on_sc=False),
        out_type=[jax.ShapeDtypeStruct((_NC * _N, _D), jnp.float32),
                  jax.ShapeDtypeStruct((_NC * _N, _DE), jnp.float32)],
        scratch_types=[
            pltpu.VMEM((_CHUNK,), jnp.int32),
            pltpu.VMEM((_CHUNK,), jnp.int32),
            pltpu.VMEM((_CHUNK, _D), jnp.float32),
            pltpu.VMEM((_CHUNK, _DE), jnp.float32),
            pltpu.VMEM_SHARED((_N, _D), jnp.float32),
            pltpu.VMEM_SHARED((_N, _DE), jnp.float32),
            pltpu.SemaphoreType.DMA,
        ],
    )(_sc_agg_body)


def _post_body(x_ref, w1_ref, w2_ref, we_ref, wout_ref, bout_ref,
               aggx_ref, agge_ref, batch_ref, bshift_ref, o_ref):
    aggx = aggx_ref[pl.ds(0, _N), :]
    agge = agge_ref[pl.ds(0, _N), :]
    for c in range(1, _NC):
        aggx = aggx + aggx_ref[pl.ds(c * _N, _N), :]
        agge = agge + agge_ref[pl.ds(c * _N, _N), :]
    h = (jnp.dot(x_ref[...], w1_ref[...], preferred_element_type=jnp.float32)
         + jnp.dot(aggx, w2_ref[...], preferred_element_type=jnp.float32)
         + jnp.dot(agge, we_ref[...], preferred_element_type=jnp.float32))
    h = jnp.maximum(h, 0.0)
    b = batch_ref[...]
    bs = bshift_ref[...]
    gid = lax.broadcasted_iota(jnp.int32, (1, _N), 1)
    keep = jnp.logical_and(b == bs, gid != _N - 1)
    gids = lax.broadcasted_iota(jnp.int32, (_B, _N), 0)
    pt = jnp.where(jnp.logical_and(gids == b, keep), 1.0, 0.0)
    hg = jnp.dot(pt, h, preferred_element_type=jnp.float32)
    cnt = jnp.sum(pt, axis=1, keepdims=True)
    hg = hg / jnp.maximum(cnt, 1.0)
    o_ref[...] = (jnp.dot(hg, wout_ref[...], preferred_element_type=jnp.float32)
                  + bout_ref[...])


def _post_call(interpret=False):
    return pl.pallas_call(
        _post_body,
        out_shape=jax.ShapeDtypeStruct((_B, _OUT), jnp.float32),
        interpret=interpret,
    )


def kernel(x, edge_index, edge_attr, batch, W1, W2, We, Wout, bout):
    src = edge_index[0].astype(jnp.int32)
    dst = edge_index[1].astype(jnp.int32)
    batch32 = batch.astype(jnp.int32)
    bshift = jnp.concatenate([batch32[:1], batch32[:-1]])
    zx = jnp.zeros((_CHUNK, _D), jnp.float32)
    ze = jnp.zeros((_CHUNK, _DE), jnp.float32)
    iota = jnp.arange(_NC * _N, dtype=jnp.int32)
    aggx, agge = _sc_agg()(x, edge_attr, src, dst, zx, ze, iota)
    return _post_call()(
        x, W1, W2, We, Wout, bout.reshape(1, _OUT),
        aggx, agge, batch32.reshape(1, _N), bshift.reshape(1, _N))


# speedup vs baseline: 5.2535x; 1.4694x over previous
import functools

import jax
import jax.numpy as jnp
from jax import lax
from jax.experimental import pallas as pl
from jax.experimental.pallas import tpu as pltpu
from jax.experimental.pallas import tpu_sc as plsc

_N = 10000
_E = 320000
_D = 128
_DE = 16
_B = 512
_OUT = 12

_NC = 2
_NS = 16
_CHUNK = 80
_EPC = _E // _NC
_EPW = _EPC // _NS
_NCHUNK = _EPW // _CHUNK
_SUP = 5
_RSTRIPE = 624
_RCHUNKS = 8


def _sc_agg_body(x_hbm, ea_hbm, src_hbm, dst_hbm, zx_hbm, ze_hbm, iota_hbm,
                 outx_hbm, oute_hbm,
                 idx_v, src5_v, dst5_v, rows0_v, rows1_v, ea5_v,
                 accx_s, acce_s, gsem0, gsem1):
    c = lax.axis_index("c")
    s = lax.axis_index("s")

    pltpu.sync_copy(zx_hbm, rows0_v)
    pltpu.sync_copy(ze_hbm, ea5_v.at[pl.ds(0, _CHUNK)])

    def zstep(j, carry):
        base = s * _RSTRIPE + j * _CHUNK
        pltpu.sync_copy(iota_hbm.at[pl.ds(base, _CHUNK)], idx_v)
        pltpu.sync_copy(rows0_v, accx_s.at[idx_v])
        pltpu.sync_copy(ea5_v.at[pl.ds(0, _CHUNK)], acce_s.at[idx_v])
        return carry

    lax.fori_loop(0, _RCHUNKS, zstep, 0)
    plsc.subcore_barrier()

    r0 = (c * _EPC + s * _EPW) // _CHUNK

    def sup(m, carry):
        row = r0 + m * _SUP
        pltpu.sync_copy(src_hbm.at[pl.ds(row, _SUP)], src5_v)
        pltpu.sync_copy(dst_hbm.at[pl.ds(row, _SUP)], dst5_v)
        pltpu.sync_copy(ea_hbm.at[pl.ds(row * _CHUNK, _SUP * _CHUNK)], ea5_v)
        dg = pltpu.async_copy(x_hbm.at[src5_v.at[0]], rows0_v, gsem0)
        for j in range(_SUP):
            rows_cur = rows0_v if j % 2 == 0 else rows1_v
            rows_nxt = rows1_v if j % 2 == 0 else rows0_v
            sem_nxt = gsem1 if j % 2 == 0 else gsem0
            dg.wait()
            if j + 1 < _SUP:
                dg = pltpu.async_copy(x_hbm.at[src5_v.at[j + 1]], rows_nxt,
                                      sem_nxt)
            pltpu.sync_copy(rows_cur, accx_s.at[dst5_v.at[j]], add=True)
            pltpu.sync_copy(ea5_v.at[pl.ds(j * _CHUNK, _CHUNK)],
                            acce_s.at[dst5_v.at[j]], add=True)
        return carry

    lax.fori_loop(0, _NCHUNK // _SUP, sup, 0)
    plsc.subcore_barrier()

    def wstep(j, carry):
        base = s * _RSTRIPE + j * _CHUNK
        pltpu.sync_copy(iota_hbm.at[pl.ds(base, _CHUNK)], idx_v)
        pltpu.async_copy(accx_s.at[idx_v], rows0_v, gsem0).wait()
        pltpu.sync_copy(rows0_v, outx_hbm.at[pl.ds(c * _N + base, _CHUNK)])
        pltpu.async_copy(acce_s.at[idx_v], ea5_v.at[pl.ds(0, _CHUNK)],
                         gsem0).wait()
        pltpu.sync_copy(ea5_v.at[pl.ds(0, _CHUNK)],
                        oute_hbm.at[pl.ds(c * _N + base, _CHUNK)])
        return carry

    lax.fori_loop(0, _RCHUNKS, wstep, 0)


@functools.cache
def _sc_agg():
    return functools.partial(
        pl.kernel,
        mesh=plsc.VectorSubcoreMesh(core_axis_name="c", subcore_axis_name="s",
                                    num_cores=_NC),
        compiler_params=pltpu.CompilerParams(use_tc_tiling_on_sc=False),
        out_type=[jax.ShapeDtypeStruct((_NC * _N, _D), jnp.float32),
                  jax.ShapeDtypeStruct((_NC * _N, _DE), jnp.float32)],
        scratch_types=[
            pltpu.VMEM((_CHUNK,), jnp.int32),
            pltpu.VMEM((_SUP, _CHUNK), jnp.int32),
            pltpu.VMEM((_SUP, _CHUNK), jnp.int32),
            pltpu.VMEM((_CHUNK, _D), jnp.float32),
            pltpu.VMEM((_CHUNK, _D), jnp.float32),
            pltpu.VMEM((_SUP * _CHUNK, _DE), jnp.float32),
            pltpu.VMEM_SHARED((_N, _D), jnp.float32),
            pltpu.VMEM_SHARED((_N, _DE), jnp.float32),
            pltpu.SemaphoreType.DMA,
            pltpu.SemaphoreType.DMA,
        ],
    )(_sc_agg_body)


def _post_body(x_ref, w1_ref, w2_ref, we_ref, wout_ref, bout_ref,
               aggx_ref, agge_ref, batch_ref, bshift_ref, o_ref):
    aggx = aggx_ref[pl.ds(0, _N), :]
    agge = agge_ref[pl.ds(0, _N), :]
    for c in range(1, _NC):
        aggx = aggx + aggx_ref[pl.ds(c * _N, _N), :]
        agge = agge + agge_ref[pl.ds(c * _N, _N), :]
    h = (jnp.dot(x_ref[...], w1_ref[...], preferred_element_type=jnp.float32)
         + jnp.dot(aggx, w2_ref[...], preferred_element_type=jnp.float32)
         + jnp.dot(agge, we_ref[...], preferred_element_type=jnp.float32))
    h = jnp.maximum(h, 0.0)
    b = batch_ref[...]
    bs = bshift_ref[...]
    gid = lax.broadcasted_iota(jnp.int32, (1, _N), 1)
    keep = jnp.logical_and(b == bs, gid != _N - 1)
    gids = lax.broadcasted_iota(jnp.int32, (_B, _N), 0)
    pt = jnp.where(jnp.logical_and(gids == b, keep), 1.0, 0.0)
    hg = jnp.dot(pt, h, preferred_element_type=jnp.float32)
    cnt = jnp.sum(pt, axis=1, keepdims=True)
    hg = hg / jnp.maximum(cnt, 1.0)
    o_ref[...] = (jnp.dot(hg, wout_ref[...], preferred_element_type=jnp.float32)
                  + bout_ref[...])


def _post_call(interpret=False):
    return pl.pallas_call(
        _post_body,
        out_shape=jax.ShapeDtypeStruct((_B, _OUT), jnp.float32),
        interpret=interpret,
    )


def kernel(x, edge_index, edge_attr, batch, W1, W2, We, Wout, bout):
    src = edge_index[0].astype(jnp.int32).reshape(_E // _CHUNK, _CHUNK)
    dst = edge_index[1].astype(jnp.int32).reshape(_E // _CHUNK, _CHUNK)
    batch32 = batch.astype(jnp.int32)
    bshift = jnp.concatenate([batch32[:1], batch32[:-1]])
    zx = jnp.zeros((_CHUNK, _D), jnp.float32)
    ze = jnp.zeros((_CHUNK, _DE), jnp.float32)
    iota = jnp.arange(_NC * _N, dtype=jnp.int32)
    aggx, agge = _sc_agg()(x, edge_attr, src, dst, zx, ze, iota)
    return _post_call()(
        x, W1, W2, We, Wout, bout.reshape(1, _OUT),
        aggx, agge, batch32.reshape(1, _N), bshift.reshape(1, _N))


# speedup vs baseline: 5.2694x; 1.0030x over previous
import functools

import jax
import jax.numpy as jnp
from jax import lax
from jax.experimental import pallas as pl
from jax.experimental.pallas import tpu as pltpu
from jax.experimental.pallas import tpu_sc as plsc

_N = 10000
_E = 320000
_D = 128
_DE = 16
_B = 512
_OUT = 12

_NC = 2
_NS = 16
_CHUNK = 80
_EPC = _E // _NC
_EPW = _EPC // _NS
_NCHUNK = _EPW // _CHUNK
_SUP = 5
_RSTRIPE = 624
_RCHUNKS = 8


def _sc_agg_body(x_hbm, ea_hbm, src_hbm, dst_hbm, zx_hbm, ze_hbm, iota_hbm,
                 outx_hbm, oute_hbm,
                 idx_v, src5_v, dst5_v, rows0_v, rows1_v, ea5_v,
                 accx_s, acce_s, gsem0, gsem1, csem0, csem1):
    c = lax.axis_index("c")
    s = lax.axis_index("s")

    pltpu.sync_copy(zx_hbm, rows0_v)
    pltpu.sync_copy(ze_hbm, ea5_v.at[pl.ds(0, _CHUNK)])

    def zstep(j, carry):
        base = s * _RSTRIPE + j * _CHUNK
        pltpu.sync_copy(iota_hbm.at[pl.ds(base, _CHUNK)], idx_v)
        pltpu.sync_copy(rows0_v, accx_s.at[idx_v])
        pltpu.sync_copy(ea5_v.at[pl.ds(0, _CHUNK)], acce_s.at[idx_v])
        return carry

    lax.fori_loop(0, _RCHUNKS, zstep, 0)
    plsc.subcore_barrier()

    r0 = (c * _EPC + s * _EPW) // _CHUNK

    def sup(m, carry):
        row = r0 + m * _SUP
        pltpu.sync_copy(src_hbm.at[pl.ds(row, _SUP)], src5_v)
        pltpu.sync_copy(dst_hbm.at[pl.ds(row, _SUP)], dst5_v)
        pltpu.sync_copy(ea_hbm.at[pl.ds(row * _CHUNK, _SUP * _CHUNK)], ea5_v)
        dg = pltpu.async_copy(x_hbm.at[src5_v.at[0]], rows0_v, gsem0)
        sc_prev = None
        for j in range(_SUP):
            rows_cur = rows0_v if j % 2 == 0 else rows1_v
            rows_nxt = rows1_v if j % 2 == 0 else rows0_v
            gsem_nxt = gsem1 if j % 2 == 0 else gsem0
            csem_cur = csem0 if j % 2 == 0 else csem1
            dg.wait()
            if sc_prev is not None:
                sc_prev[0].wait()
                sc_prev[1].wait()
            if j + 1 < _SUP:
                dg = pltpu.async_copy(x_hbm.at[src5_v.at[j + 1]], rows_nxt,
                                      gsem_nxt)
            sc_prev = (
                pltpu.async_copy(rows_cur, accx_s.at[dst5_v.at[j]], csem_cur,
                                 add=True),
                pltpu.async_copy(ea5_v.at[pl.ds(j * _CHUNK, _CHUNK)],
                                 acce_s.at[dst5_v.at[j]], csem_cur, add=True),
            )
        sc_prev[0].wait()
        sc_prev[1].wait()
        return carry

    lax.fori_loop(0, _NCHUNK // _SUP, sup, 0)
    plsc.subcore_barrier()

    def wstep(j, carry):
        base = s * _RSTRIPE + j * _CHUNK
        pltpu.sync_copy(iota_hbm.at[pl.ds(base, _CHUNK)], idx_v)
        pltpu.async_copy(accx_s.at[idx_v], rows0_v, gsem0).wait()
        pltpu.sync_copy(rows0_v, outx_hbm.at[pl.ds(c * _N + base, _CHUNK)])
        pltpu.async_copy(acce_s.at[idx_v], ea5_v.at[pl.ds(0, _CHUNK)],
                         gsem0).wait()
        pltpu.sync_copy(ea5_v.at[pl.ds(0, _CHUNK)],
                        oute_hbm.at[pl.ds(c * _N + base, _CHUNK)])
        return carry

    lax.fori_loop(0, _RCHUNKS, wstep, 0)


@functools.cache
def _sc_agg():
    return functools.partial(
        pl.kernel,
        mesh=plsc.VectorSubcoreMesh(core_axis_name="c", subcore_axis_name="s",
                                    num_cores=_NC),
        compiler_params=pltpu.CompilerParams(use_tc_tiling_on_sc=False),
        out_type=[jax.ShapeDtypeStruct((_NC * _N, _D), jnp.float32),
                  jax.ShapeDtypeStruct((_NC * _N, _DE), jnp.float32)],
        scratch_types=[
            pltpu.VMEM((_CHUNK,), jnp.int32),
            pltpu.VMEM((_SUP, _CHUNK), jnp.int32),
            pltpu.VMEM((_SUP, _CHUNK), jnp.int32),
            pltpu.VMEM((_CHUNK, _D), jnp.float32),
            pltpu.VMEM((_CHUNK, _D), jnp.float32),
            pltpu.VMEM((_SUP * _CHUNK, _DE), jnp.float32),
            pltpu.VMEM_SHARED((_N, _D), jnp.float32),
            pltpu.VMEM_SHARED((_N, _DE), jnp.float32),
            pltpu.SemaphoreType.DMA,
            pltpu.SemaphoreType.DMA,
            pltpu.SemaphoreType.DMA,
            pltpu.SemaphoreType.DMA,
        ],
    )(_sc_agg_body)


def _post_body(x_ref, w1_ref, w2_ref, we_ref, wout_ref, bout_ref,
               aggx_ref, agge_ref, batch_ref, bshift_ref, o_ref):
    aggx = aggx_ref[pl.ds(0, _N), :]
    agge = agge_ref[pl.ds(0, _N), :]
    for c in range(1, _NC):
        aggx = aggx + aggx_ref[pl.ds(c * _N, _N), :]
        agge = agge + agge_ref[pl.ds(c * _N, _N), :]
    h = (jnp.dot(x_ref[...], w1_ref[...], preferred_element_type=jnp.float32)
         + jnp.dot(aggx, w2_ref[...], preferred_element_type=jnp.float32)
         + jnp.dot(agge, we_ref[...], preferred_element_type=jnp.float32))
    h = jnp.maximum(h, 0.0)
    b = batch_ref[...]
    bs = bshift_ref[...]
    gid = lax.broadcasted_iota(jnp.int32, (1, _N), 1)
    keep = jnp.logical_and(b == bs, gid != _N - 1)
    gids = lax.broadcasted_iota(jnp.int32, (_B, _N), 0)
    pt = jnp.where(jnp.logical_and(gids == b, keep), 1.0, 0.0)
    hg = jnp.dot(pt, h, preferred_element_type=jnp.float32)
    cnt = jnp.sum(pt, axis=1, keepdims=True)
    hg = hg / jnp.maximum(cnt, 1.0)
    o_ref[...] = (jnp.dot(hg, wout_ref[...], preferred_element_type=jnp.float32)
                  + bout_ref[...])


def _post_call(interpret=False):
    return pl.pallas_call(
        _post_body,
        out_shape=jax.ShapeDtypeStruct((_B, _OUT), jnp.float32),
        interpret=interpret,
    )


def kernel(x, edge_index, edge_attr, batch, W1, W2, We, Wout, bout):
    src = edge_index[0].astype(jnp.int32).reshape(_E // _CHUNK, _CHUNK)
    dst = edge_index[1].astype(jnp.int32).reshape(_E // _CHUNK, _CHUNK)
    batch32 = batch.astype(jnp.int32)
    bshift = jnp.concatenate([batch32[:1], batch32[:-1]])
    zx = jnp.zeros((_CHUNK, _D), jnp.float32)
    ze = jnp.zeros((_CHUNK, _DE), jnp.float32)
    iota = jnp.arange(_NC * _N, dtype=jnp.int32)
    aggx, agge = _sc_agg()(x, edge_attr, src, dst, zx, ze, iota)
    return _post_call()(
        x, W1, W2, We, Wout, bout.reshape(1, _OUT),
        aggx, agge, batch32.reshape(1, _N), bshift.reshape(1, _N))


# speedup vs baseline: 5.5384x; 1.0510x over previous
import functools

import jax
import jax.numpy as jnp
from jax import lax
from jax.experimental import pallas as pl
from jax.experimental.pallas import tpu as pltpu
from jax.experimental.pallas import tpu_sc as plsc

_N = 10000
_E = 320000
_D = 128
_DE = 16
_B = 512
_OUT = 12

_NC = 2
_NS = 16
_CHUNK = 80
_EPC = _E // _NC
_EPW = _EPC // _NS
_NCHUNK = _EPW // _CHUNK
_SUP = 5
_RSTRIPE = 624
_RCHUNKS = 8


def _sc_agg_body(x_hbm, ea_hbm, src_hbm, dst_hbm, zx_hbm, ze_hbm, iota_hbm,
                 outx_hbm, oute_hbm,
                 idx_v, src5_v, dst5_v, rows0_v, rows1_v, ea5_v,
                 accx_s, acce_s, gsem0, gsem1, csem0, csem1):
    c = lax.axis_index("c")
    s = lax.axis_index("s")

    pltpu.sync_copy(zx_hbm, rows0_v)
    pltpu.sync_copy(ze_hbm, ea5_v.at[pl.ds(0, _CHUNK)])

    def zstep(j, carry):
        base = s * _RSTRIPE + j * _CHUNK
        pltpu.sync_copy(iota_hbm.at[pl.ds(base, _CHUNK)], idx_v)
        pltpu.sync_copy(rows0_v, accx_s.at[idx_v])
        pltpu.sync_copy(ea5_v.at[pl.ds(0, _CHUNK)], acce_s.at[idx_v])
        return carry

    lax.fori_loop(0, _RCHUNKS, zstep, 0)
    plsc.subcore_barrier()

    r0 = (c * _EPC + s * _EPW) // _CHUNK

    def sup(m, carry):
        row = r0 + m * _SUP
        pltpu.sync_copy(src_hbm.at[pl.ds(row, _SUP)], src5_v)
        pltpu.sync_copy(dst_hbm.at[pl.ds(row, _SUP)], dst5_v)
        pltpu.sync_copy(ea_hbm.at[pl.ds(row * _CHUNK, _SUP * _CHUNK)], ea5_v)
        dg = pltpu.async_copy(x_hbm.at[src5_v.at[0]], rows0_v, gsem0)
        sc_prev = None
        for j in range(_SUP):
            rows_cur = rows0_v if j % 2 == 0 else rows1_v
            rows_nxt = rows1_v if j % 2 == 0 else rows0_v
            gsem_nxt = gsem1 if j % 2 == 0 else gsem0
            csem_cur = csem0 if j % 2 == 0 else csem1
            if sc_prev is not None:
                sc_prev[0].wait()
                sc_prev[1].wait()
            if j + 1 < _SUP:
                dg_nxt = pltpu.async_copy(x_hbm.at[src5_v.at[j + 1]], rows_nxt,
                                          gsem_nxt)
            dg.wait()
            if j + 1 < _SUP:
                dg = dg_nxt
            sc_prev = (
                pltpu.async_copy(rows_cur, accx_s.at[dst5_v.at[j]], csem_cur,
                                 add=True),
                pltpu.async_copy(ea5_v.at[pl.ds(j * _CHUNK, _CHUNK)],
                                 acce_s.at[dst5_v.at[j]], csem_cur, add=True),
            )
        sc_prev[0].wait()
        sc_prev[1].wait()
        return carry

    lax.fori_loop(0, _NCHUNK // _SUP, sup, 0)
    plsc.subcore_barrier()

    def wstep(j, carry):
        base = s * _RSTRIPE + j * _CHUNK
        pltpu.sync_copy(iota_hbm.at[pl.ds(base, _CHUNK)], idx_v)
        pltpu.async_copy(accx_s.at[idx_v], rows0_v, gsem0).wait()
        pltpu.sync_copy(rows0_v, outx_hbm.at[pl.ds(c * _N + base, _CHUNK)])
        pltpu.async_copy(acce_s.at[idx_v], ea5_v.at[pl.ds(0, _CHUNK)],
                         gsem0).wait()
        pltpu.sync_copy(ea5_v.at[pl.ds(0, _CHUNK)],
                        oute_hbm.at[pl.ds(c * _N + base, _CHUNK)])
        return carry

    lax.fori_loop(0, _RCHUNKS, wstep, 0)


@functools.cache
def _sc_agg():
    return functools.partial(
        pl.kernel,
        mesh=plsc.VectorSubcoreMesh(core_axis_name="c", subcore_axis_name="s",
                                    num_cores=_NC),
        compiler_params=pltpu.CompilerParams(use_tc_tiling_on_sc=False),
        out_type=[jax.ShapeDtypeStruct((_NC * _N, _D), jnp.float32),
                  jax.ShapeDtypeStruct((_NC * _N, _DE), jnp.float32)],
        scratch_types=[
            pltpu.VMEM((_CHUNK,), jnp.int32),
            pltpu.VMEM((_SUP, _CHUNK), jnp.int32),
            pltpu.VMEM((_SUP, _CHUNK), jnp.int32),
            pltpu.VMEM((_CHUNK, _D), jnp.float32),
            pltpu.VMEM((_CHUNK, _D), jnp.float32),
            pltpu.VMEM((_SUP * _CHUNK, _DE), jnp.float32),
            pltpu.VMEM_SHARED((_N, _D), jnp.float32),
            pltpu.VMEM_SHARED((_N, _DE), jnp.float32),
            pltpu.SemaphoreType.DMA,
            pltpu.SemaphoreType.DMA,
            pltpu.SemaphoreType.DMA,
            pltpu.SemaphoreType.DMA,
        ],
    )(_sc_agg_body)


def _post_body(x_ref, w1_ref, w2_ref, we_ref, wout_ref, bout_ref,
               aggx_ref, agge_ref, batch_ref, bshift_ref, o_ref):
    aggx = aggx_ref[pl.ds(0, _N), :]
    agge = agge_ref[pl.ds(0, _N), :]
    for c in range(1, _NC):
        aggx = aggx + aggx_ref[pl.ds(c * _N, _N), :]
        agge = agge + agge_ref[pl.ds(c * _N, _N), :]
    h = (jnp.dot(x_ref[...], w1_ref[...], preferred_element_type=jnp.float32)
         + jnp.dot(aggx, w2_ref[...], preferred_element_type=jnp.float32)
         + jnp.dot(agge, we_ref[...], preferred_element_type=jnp.float32))
    h = jnp.maximum(h, 0.0)
    b = batch_ref[...]
    bs = bshift_ref[...]
    gid = lax.broadcasted_iota(jnp.int32, (1, _N), 1)
    keep = jnp.logical_and(b == bs, gid != _N - 1)
    gids = lax.broadcasted_iota(jnp.int32, (_B, _N), 0)
    pt = jnp.where(jnp.logical_and(gids == b, keep), 1.0, 0.0)
    hg = jnp.dot(pt, h, preferred_element_type=jnp.float32)
    cnt = jnp.sum(pt, axis=1, keepdims=True)
    hg = hg / jnp.maximum(cnt, 1.0)
    o_ref[...] = (jnp.dot(hg, wout_ref[...], preferred_element_type=jnp.float32)
                  + bout_ref[...])


def _post_call(interpret=False):
    return pl.pallas_call(
        _post_body,
        out_shape=jax.ShapeDtypeStruct((_B, _OUT), jnp.float32),
        interpret=interpret,
    )


def kernel(x, edge_index, edge_attr, batch, W1, W2, We, Wout, bout):
    src = edge_index[0].astype(jnp.int32).reshape(_E // _CHUNK, _CHUNK)
    dst = edge_index[1].astype(jnp.int32).reshape(_E // _CHUNK, _CHUNK)
    batch32 = batch.astype(jnp.int32)
    bshift = jnp.concatenate([batch32[:1], batch32[:-1]])
    zx = jnp.zeros((_CHUNK, _D), jnp.float32)
    ze = jnp.zeros((_CHUNK, _DE), jnp.float32)
    iota = jnp.arange(_NC * _N, dtype=jnp.int32)
    aggx, agge = _sc_agg()(x, edge_attr, src, dst, zx, ze, iota)
    return _post_call()(
        x, W1, W2, We, Wout, bout.reshape(1, _OUT),
        aggx, agge, batch32.reshape(1, _N), bshift.reshape(1, _N))
